# baseline (speedup 1.0000x reference)
import functools

import jax
import jax.numpy as jnp
from jax import lax
from jax.experimental import pallas as pl
from jax.experimental.pallas import tpu as pltpu
from jax.experimental.pallas import tpu_sc as plsc

_VOCAB = 30522
_D = 128
_C = 3
_B = 16384
_L = 200

_BLKV = 6144
_VPAD = 30720

_NC = 2
_NS = 16
_NW = _NC * _NS
_ROWS_PER_W = _B // _NW
_CH = 64
_NCHUNK = _ROWS_PER_W // _CH
_SG = _CH // 16


def _project_body(e_ref, w_ref, b_ref, o0_ref, o1_ref, o2_ref):
    acc = lax.dot_general(
        w_ref[...], e_ref[...],
        dimension_numbers=(((1,), (1,)), ((), ())),
        preferred_element_type=jnp.float32,
    ) * (1.0 / _L) + b_ref[...][:, 0:1]
    o0_ref[...] = acc[0]
    o1_ref[...] = acc[1]
    o2_ref[...] = acc[2]


def _project(emb_table, w8, b2d):
    plane = jax.ShapeDtypeStruct((_VPAD,), jnp.float32)
    return pl.pallas_call(
        _project_body,
        grid=(_VPAD // _BLKV,),
        in_specs=[
            pl.BlockSpec((_BLKV, _D), lambda i: (i, 0)),
            pl.BlockSpec((8, _D), lambda i: (0, 0)),
            pl.BlockSpec((8, _D), lambda i: (0, 0)),
        ],
        out_specs=[pl.BlockSpec((_BLKV,), lambda i: (i,))] * _C,
        out_shape=[plane, plane, plane],
    )(emb_table, w8, b2d)


def _sc_bag_body(p0_hbm, p1_hbm, p2_hbm, tx_hbm, out_hbm,
                 p0, p1, p2, tv, ovi,
                 psem0, psem1, psem2, sem0, sem1):
    wid = lax.axis_index("s") * _NC + lax.axis_index("c")
    base_row = wid * _ROWS_PER_W

    sems = (sem0, sem1)

    def start(ch, buf):
        row0 = base_row + ch * _CH
        return pltpu.async_copy(
            tx_hbm.at[pl.ds(row0, _CH), :],
            tv.at[pl.ds(buf * _CH, _CH), :],
            sems[buf],
        )

    pcopies = [
        pltpu.async_copy(src, dst, sem)
        for src, dst, sem in ((p0_hbm, p0, psem0),
                              (p1_hbm, p1, psem1),
                              (p2_hbm, p2, psem2))
    ]
    cp = start(0, 0)
    for c in pcopies:
        c.wait()

    iota = lax.iota(jnp.int32, 16)
    zero = jnp.zeros((16,), jnp.float32)

    for ch in range(_NCHUNK):
        nxt = start(ch + 1, (ch + 1) % 2) if ch + 1 < _NCHUNK else None
        cp.wait()
        buf = ch % 2
        for sg in range(_SG):
            rows = iota + (buf * _CH + sg * 16)

            def body(l, accs, _rows=rows):
                a0, a1, a2 = accs
                lvec = jnp.full((16,), 0, jnp.int32) + l
                tok = plsc.load_gather(tv, [_rows, lvec])
                a0 = a0 + plsc.load_gather(p0, [tok])
                a1 = a1 + plsc.load_gather(p1, [tok])
                a2 = a2 + plsc.load_gather(p2, [tok])
                return (a0, a1, a2)

            accs = lax.fori_loop(0, _L, body, (zero, zero, zero), unroll=8)
            orow = iota * _C + ((ch * _CH + sg * 16) * _C)
            for c in range(_C):
                plsc.store_scatter(ovi, [orow + c], accs[c])
        cp = nxt

    pltpu.sync_copy(ovi, out_hbm.at[pl.ds(base_row * _C, _ROWS_PER_W * _C)])


@functools.cache
def _sc_bag():
    mesh = plsc.VectorSubcoreMesh(
        core_axis_name="c", subcore_axis_name="s",
        num_cores=_NC, num_subcores=_NS,
    )
    return pl.kernel(
        _sc_bag_body,
        out_type=jax.ShapeDtypeStruct((_B * _C,), jnp.float32),
        mesh=mesh,
        compiler_params=pltpu.CompilerParams(needs_layout_passes=False),
        scratch_types=[
            pltpu.VMEM((_VPAD,), jnp.float32),
            pltpu.VMEM((_VPAD,), jnp.float32),
            pltpu.VMEM((_VPAD,), jnp.float32),
            pltpu.VMEM((2 * _CH, _L), jnp.int32),
            pltpu.VMEM((_ROWS_PER_W * _C,), jnp.float32),
            pltpu.SemaphoreType.DMA,
            pltpu.SemaphoreType.DMA,
            pltpu.SemaphoreType.DMA,
            pltpu.SemaphoreType.DMA,
            pltpu.SemaphoreType.DMA,
        ],
    )


def kernel(texts, emb_table, lin_w, lin_b):
    w8 = jnp.zeros((8, _D), jnp.float32).at[:_C].set(lin_w)
    b8 = jnp.zeros((8,), jnp.float32).at[:_C].set(lin_b * (1.0 / _L))
    b2d = jnp.broadcast_to(b8[:, None], (8, _D))
    p0, p1, p2 = _project(emb_table, w8, b2d)
    out = _sc_bag()(p0, p1, p2, texts)
    return out.reshape(_B, _C)


# speedup vs baseline: 1.3266x; 1.3266x over previous
import functools

import jax
import jax.numpy as jnp
from jax import lax
from jax.experimental import pallas as pl
from jax.experimental.pallas import tpu as pltpu
from jax.experimental.pallas import tpu_sc as plsc

_VOCAB = 30522
_D = 128
_C = 3
_B = 16384
_L = 200

_BLKV = 6144
_VPAD = 30720

_NC = 2
_NS = 16
_NW = _NC * _NS
_ROWS_PER_W = _B // _NW
_CH = 64
_NCHUNK = _ROWS_PER_W // _CH
_SG = _CH // 16


def _project_body(e_ref, w_ref, b_ref, o0_ref, o1_ref, o2_ref):
    acc = lax.dot_general(
        w_ref[...], e_ref[...],
        dimension_numbers=(((1,), (1,)), ((), ())),
        preferred_element_type=jnp.float32,
    ) * (1.0 / _L) + b_ref[...][:, 0:1]
    o0_ref[...] = acc[0]
    o1_ref[...] = acc[1]
    o2_ref[...] = acc[2]


def _project(emb_table, w8, b2d):
    plane = jax.ShapeDtypeStruct((_VPAD,), jnp.float32)
    return pl.pallas_call(
        _project_body,
        grid=(_VPAD // _BLKV,),
        in_specs=[
            pl.BlockSpec((_BLKV, _D), lambda i: (i, 0)),
            pl.BlockSpec((8, _D), lambda i: (0, 0)),
            pl.BlockSpec((8, _D), lambda i: (0, 0)),
        ],
        out_specs=[pl.BlockSpec((_BLKV,), lambda i: (i,))] * _C,
        out_shape=[plane, plane, plane],
    )(emb_table, w8, b2d)


def _sc_bag_body(p0_hbm, p1_hbm, p2_hbm, tx_hbm, out_hbm,
                 p0, p1, p2, tv, ov0, ov1, ov2,
                 psem0, psem1, psem2, sem0, sem1):
    wid = lax.axis_index("s") * _NC + lax.axis_index("c")
    base_row = wid * _ROWS_PER_W
    base_ch = wid * _NCHUNK
    chlen = _CH * _L

    sems = (sem0, sem1)

    def start(ch, buf):
        return pltpu.async_copy(
            tx_hbm.at[pl.ds(base_ch + ch, 1), :],
            tv.at[pl.ds(buf, 1), :],
            sems[buf],
        )

    pcopies = [
        pltpu.async_copy(src, dst, sem)
        for src, dst, sem in ((p0_hbm, p0, psem0),
                              (p1_hbm, p1, psem1),
                              (p2_hbm, p2, psem2))
    ]
    cp = start(0, 0)
    for c in pcopies:
        c.wait()

    lane = lax.iota(jnp.int32, 16) * _L
    zero = jnp.zeros((16,), jnp.float32)
    ovs = (ov0, ov1, ov2)

    for ch in range(_NCHUNK):
        nxt = start(ch + 1, (ch + 1) % 2) if ch + 1 < _NCHUNK else None
        cp.wait()
        buf = ch % 2
        bufvec = jnp.full((16,), buf, jnp.int32)
        for sg in range(_SG):
            sgbase = lane + sg * (16 * _L)

            def body(l, accs, _sgbase=sgbase, _bufvec=bufvec):
                a0, a1, a2 = accs
                tok = plsc.load_gather(tv, [_bufvec, _sgbase + l])
                a0 = a0 + plsc.load_gather(p0, [tok])
                a1 = a1 + plsc.load_gather(p1, [tok])
                a2 = a2 + plsc.load_gather(p2, [tok])
                return (a0, a1, a2)

            accs = lax.fori_loop(0, _L, body, (zero, zero, zero), unroll=8)
            col = ch * _CH + sg * 16
            for c in range(_C):
                ovs[c][pl.ds(col, 16)] = accs[c]
        cp = nxt

    for c in range(_C):
        pltpu.sync_copy(ovs[c], out_hbm.at[pl.ds(c * _B + base_row, _ROWS_PER_W)])


@functools.cache
def _sc_bag():
    mesh = plsc.VectorSubcoreMesh(
        core_axis_name="c", subcore_axis_name="s",
        num_cores=_NC, num_subcores=_NS,
    )
    return pl.kernel(
        _sc_bag_body,
        out_type=jax.ShapeDtypeStruct((_C * _B,), jnp.float32),
        mesh=mesh,
        compiler_params=pltpu.CompilerParams(needs_layout_passes=False),
        scratch_types=[
            pltpu.VMEM((_VPAD,), jnp.float32),
            pltpu.VMEM((_VPAD,), jnp.float32),
            pltpu.VMEM((_VPAD,), jnp.float32),
            pltpu.VMEM((2, _CH * _L), jnp.int32),
            pltpu.VMEM((_ROWS_PER_W,), jnp.float32),
            pltpu.VMEM((_ROWS_PER_W,), jnp.float32),
            pltpu.VMEM((_ROWS_PER_W,), jnp.float32),
            pltpu.SemaphoreType.DMA,
            pltpu.SemaphoreType.DMA,
            pltpu.SemaphoreType.DMA,
            pltpu.SemaphoreType.DMA,
            pltpu.SemaphoreType.DMA,
        ],
    )


def kernel(texts, emb_table, lin_w, lin_b):
    w8 = jnp.zeros((8, _D), jnp.float32).at[:_C].set(lin_w)
    b8 = jnp.zeros((8,), jnp.float32).at[:_C].set(lin_b * (1.0 / _L))
    b2d = jnp.broadcast_to(b8[:, None], (8, _D))
    p0, p1, p2 = _project(emb_table, w8, b2d)
    tx2 = texts.reshape(_B // _CH, _CH * _L)
    out = _sc_bag()(p0, p1, p2, tx2)
    out = out.reshape(_C, _B)
    return jnp.stack([out[0], out[1], out[2]], axis=1)


# speedup vs baseline: 1.4310x; 1.0787x over previous
import functools

import jax
import jax.numpy as jnp
from jax import lax
from jax.experimental import pallas as pl
from jax.experimental.pallas import tpu as pltpu
from jax.experimental.pallas import tpu_sc as plsc

_VOCAB = 30522
_D = 128
_C = 3
_B = 16384
_L = 200

_BLKV = 6144
_VPAD = 30720

_NC = 2
_NS = 16
_NW = _NC * _NS
_ROWS_PER_W = _B // _NW
_CH = 64
_NCHUNK = _ROWS_PER_W // _CH
_SG = _CH // 16


def _project_body(e_ref, w_ref, b_ref, o01_ref, o2_ref):
    acc = lax.dot_general(
        w_ref[...], e_ref[...],
        dimension_numbers=(((1,), (1,)), ((), ())),
        preferred_element_type=jnp.float32,
    ) * (1.0 / _L) + b_ref[...][:, 0:1]
    u0 = lax.bitcast_convert_type(
        acc[0].astype(jnp.bfloat16), jnp.uint16).astype(jnp.uint32)
    u1 = lax.bitcast_convert_type(
        acc[1].astype(jnp.bfloat16), jnp.uint16).astype(jnp.uint32)
    o01_ref[...] = lax.bitcast_convert_type(
        u0 | lax.shift_left(u1, jnp.uint32(16)), jnp.int32)
    o2_ref[...] = acc[2]


def _project(emb_table, w8, b2d):
    return pl.pallas_call(
        _project_body,
        grid=(_VPAD // _BLKV,),
        in_specs=[
            pl.BlockSpec((_BLKV, _D), lambda i: (i, 0)),
            pl.BlockSpec((8, _D), lambda i: (0, 0)),
            pl.BlockSpec((8, _D), lambda i: (0, 0)),
        ],
        out_specs=[pl.BlockSpec((_BLKV,), lambda i: (i,))] * 2,
        out_shape=[jax.ShapeDtypeStruct((_VPAD,), jnp.int32),
                   jax.ShapeDtypeStruct((_VPAD,), jnp.float32)],
    )(emb_table, w8, b2d)


def _sc_bag_body(p01_hbm, p2_hbm, tx_hbm, out_hbm,
                 p01, p2, tv, ov0, ov1, ov2,
                 psem0, psem1, sem0, sem1):
    wid = lax.axis_index("s") * _NC + lax.axis_index("c")
    base_row = wid * _ROWS_PER_W
    chlen = _CH * _L
    base_ch = wid * _NCHUNK

    sems = (sem0, sem1)

    def start(ch, buf):
        return pltpu.async_copy(
            tx_hbm.at[pl.ds(base_ch + ch, 1), :],
            tv.at[pl.ds(buf, 1), :],
            sems[buf],
        )

    pcopies = [
        pltpu.async_copy(src, dst, sem)
        for src, dst, sem in ((p01_hbm, p01, psem0),
                              (p2_hbm, p2, psem1))
    ]
    cp = start(0, 0)
    for c in pcopies:
        c.wait()

    lane = lax.iota(jnp.int32, 16) * _L
    zero = jnp.zeros((16,), jnp.float32)
    ovs = (ov0, ov1, ov2)

    for ch in range(_NCHUNK):
        nxt = start(ch + 1, (ch + 1) % 2) if ch + 1 < _NCHUNK else None
        cp.wait()
        buf = ch % 2
        bufvec = jnp.full((16,), buf, jnp.int32)
        for sg in range(_SG):
            sgbase = lane + sg * (16 * _L)

            def body(l, accs, _sgbase=sgbase, _bufvec=bufvec):
                a0, a1, a2 = accs
                tok = plsc.load_gather(tv, [_bufvec, _sgbase + l])
                w01 = plsc.load_gather(p01, [tok])
                v0, v1 = plsc.unpack(
                    plsc.bitcast(w01, jnp.bfloat16),
                    format=plsc.PackFormat.INTERLEAVED,
                )
                a0 = a0 + v0
                a1 = a1 + v1
                a2 = a2 + plsc.load_gather(p2, [tok])
                return (a0, a1, a2)

            accs = lax.fori_loop(0, _L, body, (zero, zero, zero), unroll=8)
            col = ch * _CH + sg * 16
            for c in range(_C):
                ovs[c][pl.ds(col, 16)] = accs[c]
        cp = nxt

    for c in range(_C):
        pltpu.sync_copy(ovs[c], out_hbm.at[pl.ds(c * _B + base_row, _ROWS_PER_W)])


@functools.cache
def _sc_bag():
    mesh = plsc.VectorSubcoreMesh(
        core_axis_name="c", subcore_axis_name="s",
        num_cores=_NC, num_subcores=_NS,
    )
    return pl.kernel(
        _sc_bag_body,
        out_type=jax.ShapeDtypeStruct((_C * _B,), jnp.float32),
        mesh=mesh,
        compiler_params=pltpu.CompilerParams(needs_layout_passes=False),
        scratch_types=[
            pltpu.VMEM((_VPAD,), jnp.int32),
            pltpu.VMEM((_VPAD,), jnp.float32),
            pltpu.VMEM((2, _CH * _L), jnp.int32),
            pltpu.VMEM((_ROWS_PER_W,), jnp.float32),
            pltpu.VMEM((_ROWS_PER_W,), jnp.float32),
            pltpu.VMEM((_ROWS_PER_W,), jnp.float32),
            pltpu.SemaphoreType.DMA,
            pltpu.SemaphoreType.DMA,
            pltpu.SemaphoreType.DMA,
            pltpu.SemaphoreType.DMA,
        ],
    )


def kernel(texts, emb_table, lin_w, lin_b):
    w8 = jnp.zeros((8, _D), jnp.float32).at[:_C].set(lin_w)
    b8 = jnp.zeros((8,), jnp.float32).at[:_C].set(lin_b * (1.0 / _L))
    b2d = jnp.broadcast_to(b8[:, None], (8, _D))
    p01, p2 = _project(emb_table, w8, b2d)
    tx2 = texts.reshape(_B // _CH, _CH * _L)
    out = _sc_bag()(p01, p2, tx2)
    out = out.reshape(_C, _B)
    return jnp.stack([out[0], out[1], out[2]], axis=1)
